# baseline (speedup 1.0000x reference)
import functools

import jax
import jax.numpy as jnp
import numpy as np
from jax.experimental import pallas as pl
from jax.experimental.pallas import tpu as pltpu

_B = 2
_DEPTH = 6
_AHEAD = 4


def _fused_kernel(x_hbm, w_ref, gamma_ref, beta_ref, o_hbm,
                  x_buf, o_buf, y_scr, in_sem, out_sem, *, n, eps):
    g = n // _B
    m = float(n * x_hbm.shape[2])

    def start_in(i):
        pltpu.make_async_copy(x_hbm.at[pl.ds(i * _B, _B)],
                              x_buf.at[i % _DEPTH],
                              in_sem.at[i % _DEPTH]).start()

    def wait_in(i):
        pltpu.make_async_copy(x_hbm.at[pl.ds(i * _B, _B)],
                              x_buf.at[i % _DEPTH],
                              in_sem.at[i % _DEPTH]).wait()

    def start_out(i):
        pltpu.make_async_copy(o_buf.at[i % _DEPTH],
                              o_hbm.at[pl.ds(i * _B, _B)],
                              out_sem.at[i % _DEPTH]).start()

    def wait_out(i):
        pltpu.make_async_copy(o_buf.at[i % _DEPTH],
                              o_hbm.at[pl.ds(i * _B, _B)],
                              out_sem.at[i % _DEPTH]).wait()

    w = w_ref[...].astype(jnp.bfloat16)

    for i in range(min(_AHEAD, g)):
        start_in(i)
    s = None
    s2 = None
    for i in range(g):
        wait_in(i)
        if i + _AHEAD < g:
            start_in(i + _AHEAD)
        for j in range(_B):
            x = x_buf[i % _DEPTH, j].astype(jnp.bfloat16)
            y = jax.lax.dot_general(
                w, x, (((1,), (0,)), ((), ())),
                preferred_element_type=jnp.float32)
            y_scr[i * _B + j] = y.astype(jnp.bfloat16)
            si = jnp.sum(y, axis=1, keepdims=True)
            s2i = jnp.sum(y * y, axis=1, keepdims=True)
            s = si if s is None else s + si
            s2 = s2i if s2 is None else s2 + s2i

    mean = s / m
    var = jnp.maximum(s2 / m - mean * mean, 0.0)
    scale = gamma_ref[...] * jax.lax.rsqrt(var + eps)
    shift = beta_ref[...] - mean * scale

    for i in range(g):
        if i >= _AHEAD:
            wait_out(i - _AHEAD)
        for j in range(_B):
            z = y_scr[i * _B + j].astype(jnp.float32) * scale + shift
            half_z = 0.5 * z
            o_buf[i % _DEPTH, j] = (half_z * jnp.tanh(half_z)
                                    + half_z).astype(o_buf.dtype)
        start_out(i)
    for i in range(max(g - _AHEAD, 0), g):
        wait_out(i)


def kernel(x_nchw, w_oihw, gamma, beta, eps=1e-5):
    N, Cin, H, W = x_nchw.shape
    Cout = w_oihw.shape[0]
    HW = H * W

    x3 = x_nchw.reshape(N, Cin, HW)
    w2d = w_oihw.reshape(Cout, Cin)
    g2 = gamma.reshape(Cout, 1)
    b2 = beta.reshape(Cout, 1)

    out3 = pl.pallas_call(
        functools.partial(_fused_kernel, n=N, eps=eps),
        out_shape=jax.ShapeDtypeStruct((N, Cout, HW), x_nchw.dtype),
        in_specs=[
            pl.BlockSpec(memory_space=pltpu.MemorySpace.HBM),
            pl.BlockSpec(memory_space=pltpu.MemorySpace.VMEM),
            pl.BlockSpec(memory_space=pltpu.MemorySpace.VMEM),
            pl.BlockSpec(memory_space=pltpu.MemorySpace.VMEM),
        ],
        out_specs=pl.BlockSpec(memory_space=pltpu.MemorySpace.HBM),
        scratch_shapes=[
            pltpu.VMEM((_DEPTH, _B, Cin, HW), x_nchw.dtype),
            pltpu.VMEM((_DEPTH, _B, Cout, HW), x_nchw.dtype),
            pltpu.VMEM((N, Cout, HW), jnp.bfloat16),
            pltpu.SemaphoreType.DMA((_DEPTH,)),
            pltpu.SemaphoreType.DMA((_DEPTH,)),
        ],
        compiler_params=pltpu.CompilerParams(
            vmem_limit_bytes=61_000_000,
        ),
        cost_estimate=pl.CostEstimate(
            flops=int(2 * Cout * Cin * N * HW + 6 * Cout * N * HW),
            transcendentals=int(Cout * N * HW),
            bytes_accessed=int(N * HW * (Cin + Cout) * 4),
        ),
    )(x3, w2d, g2, b2)

    return out3.reshape(N, Cout, H, W)


# speedup vs baseline: 1.0185x; 1.0185x over previous
import functools

import jax
import jax.numpy as jnp
import numpy as np
from jax.experimental import pallas as pl
from jax.experimental.pallas import tpu as pltpu

_B = 2
_DEPTH = 6
_AHEAD = 4


def _fused_kernel(x_hbm, w_ref, gb_ref, o_hbm,
                  x_buf, o_buf, y_scr, in_sem, out_sem, *, n, eps):
    g = n // _B
    m = float(n * x_hbm.shape[2])

    def start_in(i):
        pltpu.make_async_copy(x_hbm.at[pl.ds(i * _B, _B)],
                              x_buf.at[i % _DEPTH],
                              in_sem.at[i % _DEPTH]).start()

    def wait_in(i):
        pltpu.make_async_copy(x_hbm.at[pl.ds(i * _B, _B)],
                              x_buf.at[i % _DEPTH],
                              in_sem.at[i % _DEPTH]).wait()

    def start_out(i):
        pltpu.make_async_copy(o_buf.at[i % _DEPTH],
                              o_hbm.at[pl.ds(i * _B, _B)],
                              out_sem.at[i % _DEPTH]).start()

    def wait_out(i):
        pltpu.make_async_copy(o_buf.at[i % _DEPTH],
                              o_hbm.at[pl.ds(i * _B, _B)],
                              out_sem.at[i % _DEPTH]).wait()

    w = w_ref[...]

    for i in range(min(_AHEAD, g)):
        start_in(i)
    s = None
    s2 = None
    for i in range(g):
        wait_in(i)
        if i + _AHEAD < g:
            start_in(i + _AHEAD)
        for j in range(_B):
            x = x_buf[i % _DEPTH, j].astype(jnp.bfloat16)
            y = jax.lax.dot_general(
                w, x, (((1,), (0,)), ((), ())),
                preferred_element_type=jnp.float32)
            y_scr[i * _B + j] = y.astype(jnp.bfloat16)
            si = jnp.sum(y, axis=1, keepdims=True)
            s2i = jnp.sum(y * y, axis=1, keepdims=True)
            s = si if s is None else s + si
            s2 = s2i if s2 is None else s2 + s2i

    mean = s / m
    var = jnp.maximum(s2 / m - mean * mean, 0.0)
    scale = gb_ref[:, 0:1] * jax.lax.rsqrt(var + eps)
    shift = gb_ref[:, 1:2] - mean * scale

    for i in range(g):
        if i >= _AHEAD:
            wait_out(i - _AHEAD)
        for j in range(_B):
            z = y_scr[i * _B + j].astype(jnp.float32) * scale + shift
            half_z = 0.5 * z
            o_buf[i % _DEPTH, j] = (half_z * jnp.tanh(half_z)
                                    + half_z).astype(o_buf.dtype)
        start_out(i)
    for i in range(max(g - _AHEAD, 0), g):
        wait_out(i)


def kernel(x_nchw, w_oihw, gamma, beta, eps=1e-5):
    N, Cin, H, W = x_nchw.shape
    Cout = w_oihw.shape[0]
    HW = H * W

    x3 = x_nchw.reshape(N, Cin, HW)
    w_bf = w_oihw.reshape(Cout, Cin).astype(jnp.bfloat16)
    gb = jnp.concatenate(
        [gamma.astype(jnp.float32).reshape(Cout, 1),
         beta.astype(jnp.float32).reshape(Cout, 1)], axis=1)

    out3 = pl.pallas_call(
        functools.partial(_fused_kernel, n=N, eps=eps),
        out_shape=jax.ShapeDtypeStruct((N, Cout, HW), x_nchw.dtype),
        in_specs=[
            pl.BlockSpec(memory_space=pltpu.MemorySpace.HBM),
            pl.BlockSpec(memory_space=pltpu.MemorySpace.VMEM),
            pl.BlockSpec(memory_space=pltpu.MemorySpace.VMEM),
        ],
        out_specs=pl.BlockSpec(memory_space=pltpu.MemorySpace.HBM),
        scratch_shapes=[
            pltpu.VMEM((_DEPTH, _B, Cin, HW), x_nchw.dtype),
            pltpu.VMEM((_DEPTH, _B, Cout, HW), x_nchw.dtype),
            pltpu.VMEM((N, Cout, HW), jnp.bfloat16),
            pltpu.SemaphoreType.DMA((_DEPTH,)),
            pltpu.SemaphoreType.DMA((_DEPTH,)),
        ],
        compiler_params=pltpu.CompilerParams(
            vmem_limit_bytes=61_000_000,
        ),
        cost_estimate=pl.CostEstimate(
            flops=int(2 * Cout * Cin * N * HW + 6 * Cout * N * HW),
            transcendentals=int(Cout * N * HW),
            bytes_accessed=int(N * HW * (Cin + Cout) * 4),
        ),
    )(x3, w_bf, gb)

    return out3.reshape(N, Cout, H, W)
